# baseline (speedup 1.0000x reference)
import functools

import jax
import jax.numpy as jnp
from jax import lax
from jax.experimental import pallas as pl
from jax.experimental.pallas import tpu as pltpu
from jax.experimental.pallas import tpu_sc as plsc

_N, _E, _DIN, _DH, _DE = 10000, 320000, 128, 256, 16
_NLAYERS, _NGROUP, _NCLS = 3, 16, 10

_INTERPRET = False
_USE_SC = True

_SC_NC, _SC_NS, _L = 2, 16, 16

_RPT = 624
_RTAIL = _N - _SC_NS * _RPT
_K = 128


def _sigmoid(x):
    return 1.0 / (1.0 + jnp.exp(-x))


def _edge_enc_body(ea_ref, wt_ref, bt_ref, g_ref, be_ref, we_ref, bex_ref,
                   t_ref, ex0_ref, ex1_ref, ex2_ref):
    ea = ea_ref[...]
    t = ea @ wt_ref[...] + bt_ref[...]
    t = jnp.maximum(t, 0.0)
    m = jnp.mean(t, axis=-1, keepdims=True)
    v = jnp.mean((t - m) ** 2, axis=-1, keepdims=True)
    t = (t - m) * lax.rsqrt(v + 1e-5) * g_ref[...] + be_ref[...]
    t_ref[...] = t
    ex = ea @ we_ref[...] + bex_ref[...]
    for l, ref in enumerate((ex0_ref, ex1_ref, ex2_ref)):
        ref[0] = ex[:, 256 * l:256 * l + 128]
        ref[1] = ex[:, 256 * l + 128:256 * l + 256]


def _edge_enc(edge_attr, wt, bt, g, be, we, bex):
    blk = 1280
    grid = _E // blk
    f32 = jnp.float32
    return pl.pallas_call(
        _edge_enc_body,
        grid=(grid,),
        in_specs=[
            pl.BlockSpec((blk, _DE), lambda i: (i, 0)),
            pl.BlockSpec((_DE, _DIN), lambda i: (0, 0)),
            pl.BlockSpec((1, _DIN), lambda i: (0, 0)),
            pl.BlockSpec((1, _DIN), lambda i: (0, 0)),
            pl.BlockSpec((1, _DIN), lambda i: (0, 0)),
            pl.BlockSpec((_DE, 3 * _DH), lambda i: (0, 0)),
            pl.BlockSpec((1, 3 * _DH), lambda i: (0, 0)),
        ],
        out_specs=[
            pl.BlockSpec((blk, 128), lambda i: (i, 0)),
            pl.BlockSpec((2, blk, 128), lambda i: (0, i, 0)),
            pl.BlockSpec((2, blk, 128), lambda i: (0, i, 0)),
            pl.BlockSpec((2, blk, 128), lambda i: (0, i, 0)),
        ],
        out_shape=[
            jax.ShapeDtypeStruct((_E, 128), f32),
            jax.ShapeDtypeStruct((2, _E, 128), f32),
            jax.ShapeDtypeStruct((2, _E, 128), f32),
            jax.ShapeDtypeStruct((2, _E, 128), f32),
        ],
        interpret=_INTERPRET,
    )(edge_attr, wt, bt, g, be, we, bex)


def _emb_body(x_ref, nfa_ref, w_ref, b_ref, h_ref):
    nf = nfa_ref[0]
    deg = nfa_ref[1, :, 0:1]
    hin = x_ref[...] + nf / jnp.maximum(deg, 1.0)
    h_ref[...] = jnp.dot(hin, w_ref[...],
                         preferred_element_type=jnp.float32) + b_ref[...]


def _emb(x, nfa, w, b):
    blk = 1000
    return pl.pallas_call(
        _emb_body,
        grid=(_N // blk,),
        in_specs=[
            pl.BlockSpec((blk, _DIN), lambda i: (i, 0)),
            pl.BlockSpec((2, blk, 128), lambda i: (0, i, 0)),
            pl.BlockSpec((_DIN, _DH), lambda i: (0, 0)),
            pl.BlockSpec((1, _DH), lambda i: (0, 0)),
        ],
        out_specs=pl.BlockSpec((blk, _DH), lambda i: (i, 0)),
        out_shape=jax.ShapeDtypeStruct((_N, _DH), jnp.float32),
        interpret=_INTERPRET,
    )(x, nfa, w, b)


def _node_mm_body(h_ref, w_ref, b_ref, ab_ref, ct_ref, dx_ref):
    p = jnp.dot(h_ref[...], w_ref[...],
                preferred_element_type=jnp.float32) + b_ref[...]
    ab_ref[0] = p[:, 0:256]
    ab_ref[1] = p[:, 256:512]
    ct_ref[0] = p[:, 512:640]
    ct_ref[1] = p[:, 640:768]
    dx_ref[...] = p[:, 768:1024]


def _node_mm(h, w, b):
    blk = 1000
    f32 = jnp.float32
    return pl.pallas_call(
        _node_mm_body,
        grid=(_N // blk,),
        in_specs=[
            pl.BlockSpec((blk, _DH), lambda i: (i, 0)),
            pl.BlockSpec((_DH, 4 * _DH), lambda i: (0, 0)),
            pl.BlockSpec((1, 4 * _DH), lambda i: (0, 0)),
        ],
        out_specs=[
            pl.BlockSpec((2, blk, 256), lambda i: (0, i, 0)),
            pl.BlockSpec((2, blk, 128), lambda i: (0, i, 0)),
            pl.BlockSpec((blk, _DH), lambda i: (i, 0)),
        ],
        out_shape=[
            jax.ShapeDtypeStruct((2, _N, 256), f32),
            jax.ShapeDtypeStruct((2, _N, 128), f32),
            jax.ShapeDtypeStruct((_N, _DH), f32),
        ],
        interpret=_INTERPRET,
    )(h, w, b)


def _bn_stats_body(agg_ref, dx_ref, h_ref, u_ref, st_ref):
    agg = jnp.concatenate([agg_ref[0], agg_ref[1]], axis=-1)
    u = agg * _sigmoid(dx_ref[...]) + h_ref[...]
    u_ref[...] = u
    s1 = jnp.sum(u, axis=0, keepdims=True)
    s2 = jnp.sum(u * u, axis=0, keepdims=True)
    blk = jnp.concatenate([s1, s2, jnp.zeros((6, _DH), jnp.float32)], axis=0)

    @pl.when(pl.program_id(0) == 0)
    def _():
        st_ref[...] = blk

    @pl.when(pl.program_id(0) != 0)
    def _():
        st_ref[...] += blk


def _bn_stats(agg2, dx, h):
    blk = 1000
    f32 = jnp.float32
    return pl.pallas_call(
        _bn_stats_body,
        grid=(_N // blk,),
        in_specs=[
            pl.BlockSpec((2, blk, 128), lambda i: (0, i, 0)),
            pl.BlockSpec((blk, _DH), lambda i: (i, 0)),
            pl.BlockSpec((blk, _DH), lambda i: (i, 0)),
        ],
        out_specs=[
            pl.BlockSpec((blk, _DH), lambda i: (i, 0)),
            pl.BlockSpec((8, _DH), lambda i: (0, 0)),
        ],
        out_shape=[
            jax.ShapeDtypeStruct((_N, _DH), f32),
            jax.ShapeDtypeStruct((8, _DH), f32),
        ],
        interpret=_INTERPRET,
    )(agg2, dx, h)


def _bn_apply_body(u_ref, st_ref, g_ref, b_ref, h_ref, h2_ref):
    m = st_ref[0:1] / _N
    var = st_ref[1:2] / _N - m * m
    rstd = lax.rsqrt(var + 1e-5)
    hh = (u_ref[...] - m) * rstd * g_ref[...] + b_ref[...]
    hh = jnp.maximum(hh, 0.0)
    h_ref[...] = hh
    h2_ref[0] = hh[:, :128]
    h2_ref[1] = hh[:, 128:]


def _bn_apply(u, st, g, b):
    blk = 1000
    f32 = jnp.float32
    return pl.pallas_call(
        _bn_apply_body,
        grid=(_N // blk,),
        in_specs=[
            pl.BlockSpec((blk, _DH), lambda i: (i, 0)),
            pl.BlockSpec((8, _DH), lambda i: (0, 0)),
            pl.BlockSpec((1, _DH), lambda i: (0, 0)),
            pl.BlockSpec((1, _DH), lambda i: (0, 0)),
        ],
        out_specs=[
            pl.BlockSpec((blk, _DH), lambda i: (i, 0)),
            pl.BlockSpec((2, blk, 128), lambda i: (0, i, 0)),
        ],
        out_shape=[
            jax.ShapeDtypeStruct((_N, _DH), f32),
            jax.ShapeDtypeStruct((2, _N, 128), f32),
        ],
        interpret=_INTERPRET,
    )(u, st, g, b)


def _dec_mlp_body(d_ref, w1_ref, b1_ref, w2_ref, b2_ref, o_ref):
    d = jnp.concatenate([d_ref[0], d_ref[1]], axis=-1)
    z = jnp.dot(d, w1_ref[...], preferred_element_type=jnp.float32)
    z = jnp.maximum(z + b1_ref[...], 0.0)
    ep = jnp.dot(z, w2_ref[...], preferred_element_type=jnp.float32)
    o_ref[...] = _sigmoid(ep + b2_ref[...])


def _dec_mlp(dcat, w1, b1, w2, b2):
    blk = 1280
    return pl.pallas_call(
        _dec_mlp_body,
        grid=(_E // blk,),
        in_specs=[
            pl.BlockSpec((2, blk, 128), lambda i: (0, i, 0)),
            pl.BlockSpec((_DH, _DH), lambda i: (0, 0)),
            pl.BlockSpec((1, _DH), lambda i: (0, 0)),
            pl.BlockSpec((_DH, 1), lambda i: (0, 0)),
            pl.BlockSpec((1, 1), lambda i: (0, 0)),
        ],
        out_specs=pl.BlockSpec((blk, 1), lambda i: (i, 0)),
        out_shape=jax.ShapeDtypeStruct((_E, 1), jnp.float32),
        interpret=_INTERPRET,
    )(dcat, w1, b1, w2, b2)


def _pool_body(h_ref, bt_ref, gs_ref, gc_ref):
    b = h_ref.shape[0]
    grp = jnp.broadcast_to(bt_ref[...], (b, _NGROUP))
    iota = lax.broadcasted_iota(jnp.int32, (b, _NGROUP), 1)
    onehot = (grp == iota).astype(jnp.float32)
    gs = lax.dot_general(onehot, h_ref[...],
                         (((0,), (0,)), ((), ())),
                         preferred_element_type=jnp.float32)
    cnt = jnp.sum(onehot, axis=0)[:, None]
    gc = jnp.broadcast_to(cnt, (_NGROUP, _DH))

    @pl.when(pl.program_id(0) == 0)
    def _():
        gs_ref[...] = gs
        gc_ref[...] = gc

    @pl.when(pl.program_id(0) != 0)
    def _():
        gs_ref[...] += gs
        gc_ref[...] += gc


def _pool(h, batch2d):
    blk = 1000
    f32 = jnp.float32
    return pl.pallas_call(
        _pool_body,
        grid=(_N // blk,),
        in_specs=[
            pl.BlockSpec((blk, _DH), lambda i: (i, 0)),
            pl.BlockSpec((blk, 1), lambda i: (i, 0)),
        ],
        out_specs=[
            pl.BlockSpec((_NGROUP, _DH), lambda i: (0, 0)),
            pl.BlockSpec((_NGROUP, _DH), lambda i: (0, 0)),
        ],
        out_shape=[
            jax.ShapeDtypeStruct((_NGROUP, _DH), f32),
            jax.ShapeDtypeStruct((_NGROUP, _DH), f32),
        ],
        interpret=_INTERPRET,
    )(h, batch2d)


def _cls_body(gs_ref, gc_ref, w1_ref, b1_ref, w2_ref, b2_ref, o_ref):
    gemb = gs_ref[...] / jnp.maximum(gc_ref[...], 1.0)
    z = jnp.dot(gemb, w1_ref[...], preferred_element_type=jnp.float32)
    z = jnp.maximum(z + b1_ref[...], 0.0)
    o_ref[...] = jnp.dot(z, w2_ref[...],
                         preferred_element_type=jnp.float32) + b2_ref[...]


def _cls(gs, gc, w1, b1, w2, b2):
    return pl.pallas_call(
        _cls_body,
        out_shape=jax.ShapeDtypeStruct((_NGROUP, _NCLS), jnp.float32),
        interpret=_INTERPRET,
    )(gs, gc, w1, b1, w2, b2)


def _iota_add(dst_ref, src_ref, off):
    k = dst_ref.shape[0]
    for i in range(k // _L):
        sl = pl.ds(i * _L, _L)
        dst_ref[sl] = src_ref[sl] + off


_ENC_EPT = _E // _SC_NS
_ENC_NCH = _ENC_EPT // _K
_ENC_TAIL = _ENC_EPT - _ENC_NCH * _K


def _sc_enc_body(t_hbm, src_hbm, dst_hbm, z_hbm, out_hbm,
                 tbuf, tbuf_t, sbuf, dbuf, sbt, dbt, agg):
    c = lax.axis_index("c")
    s = lax.axis_index("s")
    r0 = s * _RPT
    pltpu.sync_copy(z_hbm.at[pl.ds(r0, _RPT)], agg.at[pl.ds(r0, _RPT)])

    @pl.when(s == 0)
    def _():
        sl = pl.ds(_SC_NS * _RPT, _RTAIL)
        pltpu.sync_copy(z_hbm.at[sl], agg.at[sl])

    @pl.when(c == 1)
    def _():
        io = lax.iota(jnp.int32, _L)
        e1 = jnp.where(io == 0, 1.0, 0.0).astype(jnp.float32)
        zv = jnp.zeros((_L,), jnp.float32)

        def fill(r, carry):
            tbuf[r, pl.ds(0, _L)] = e1
            for f in range(1, 8):
                tbuf[r, pl.ds(f * _L, _L)] = zv
            return carry

        lax.fori_loop(0, _K, fill, 0)

        def fill_t(r, carry):
            tbuf_t[r, pl.ds(0, _L)] = e1
            for f in range(1, 8):
                tbuf_t[r, pl.ds(f * _L, _L)] = zv
            return carry

        lax.fori_loop(0, _ENC_TAIL, fill_t, 0)

    plsc.subcore_barrier()

    base = s * _ENC_EPT

    def body(j, carry):
        e0 = base + j * _K
        pltpu.sync_copy(src_hbm.at[pl.ds(e0, _K)], sbuf)
        pltpu.sync_copy(dst_hbm.at[pl.ds(e0, _K)], dbuf)

        @pl.when(c == 0)
        def _():
            pltpu.sync_copy(t_hbm.at[pl.ds(e0, _K)], tbuf)

        pltpu.sync_copy(tbuf, agg.at[dbuf], add=True)
        pltpu.sync_copy(tbuf, agg.at[sbuf], add=True)
        return carry

    lax.fori_loop(0, _ENC_NCH, body, 0)
    e0 = base + _ENC_NCH * _K
    pltpu.sync_copy(src_hbm.at[pl.ds(e0, _ENC_TAIL)], sbt)
    pltpu.sync_copy(dst_hbm.at[pl.ds(e0, _ENC_TAIL)], dbt)

    @pl.when(c == 0)
    def _():
        pltpu.sync_copy(t_hbm.at[pl.ds(e0, _ENC_TAIL)], tbuf_t)

    pltpu.sync_copy(tbuf_t, agg.at[dbt], add=True)
    pltpu.sync_copy(tbuf_t, agg.at[sbt], add=True)

    plsc.subcore_barrier()
    pltpu.sync_copy(agg.at[pl.ds(r0, _RPT)], out_hbm.at[c, pl.ds(r0, _RPT)])

    @pl.when(s == 0)
    def _():
        sl = pl.ds(_SC_NS * _RPT, _RTAIL)
        pltpu.sync_copy(agg.at[sl], out_hbm.at[c, sl])


_KM = 64
_MSG_EPT = _E // _SC_NS
_MSG_NCH = _MSG_EPT // _KM
_MSG_TAIL = _MSG_EPT - _MSG_NCH * _KM


def _sc_msg_body(ab_hbm, ct_hbm, ex_hbm, src_hbm, dst_hbm, z_hbm, out_hbm,
                 sbuf, dbuf, s2buf, d2buf, abbuf, cbuf, exbuf,
                 sbt, dbt, s2t, d2t, agg, sem0, sem1):
    c = lax.axis_index("c")
    s = lax.axis_index("s")
    cn = c * _N
    r0 = s * _RPT
    pltpu.sync_copy(z_hbm.at[pl.ds(r0, _RPT)], agg.at[pl.ds(r0, _RPT)])

    @pl.when(s == 0)
    def _():
        sl = pl.ds(_SC_NS * _RPT, _RTAIL)
        pltpu.sync_copy(z_hbm.at[sl], agg.at[sl])

    plsc.subcore_barrier()

    base = s * _MSG_EPT

    def chunk(e0, k, sb, db, s2b, d2b, abb, cb, exb):
        pltpu.sync_copy(src_hbm.at[pl.ds(e0, k)], sb)
        pltpu.sync_copy(dst_hbm.at[pl.ds(e0, k)], db)
        _iota_add(s2b, sb, cn)
        _iota_add(d2b, db, cn)
        cp0 = pltpu.async_copy(ab_hbm.at[s2b], abb, sem0)
        cp1 = pltpu.async_copy(ct_hbm.at[d2b], cb, sem1)
        pltpu.sync_copy(ex_hbm.at[c, pl.ds(e0, k)], exb)
        cp0.wait()
        cp1.wait()

        def row(r, carry):
            for f in range(8):
                sl = pl.ds(f * _L, _L)
                a = abb[r, sl]
                bb = abb[r, pl.ds(128 + f * _L, _L)]
                g = bb + cb[r, sl] + exb[r, sl]
                sg = 1.0 / (1.0 + jnp.exp(-g))
                exb[r, sl] = a * sg
            return carry

        lax.fori_loop(0, k, row, 0)
        pltpu.sync_copy(exb, agg.at[db], add=True)

    def body(j, carry):
        chunk(base + j * _KM, _KM, sbuf, dbuf, s2buf, d2buf,
              abbuf, cbuf, exbuf)
        return carry

    lax.fori_loop(0, _MSG_NCH, body, 0)
    tl = pl.ds(0, _MSG_TAIL)
    chunk(base + _MSG_NCH * _KM, _MSG_TAIL, sbt, dbt, s2t, d2t,
          abbuf.at[tl], cbuf.at[tl], exbuf.at[tl])

    plsc.subcore_barrier()
    pltpu.sync_copy(agg.at[pl.ds(r0, _RPT)], out_hbm.at[c, pl.ds(r0, _RPT)])

    @pl.when(s == 0)
    def _():
        sl = pl.ds(_SC_NS * _RPT, _RTAIL)
        pltpu.sync_copy(agg.at[sl], out_hbm.at[c, sl])


_DEC_EPT = _E // _SC_NS
_DEC_NCH = _DEC_EPT // _K
_DEC_TAIL = _DEC_EPT - _DEC_NCH * _K


def _sc_dec_body(h2_hbm, src_hbm, dst_hbm, out_hbm,
                 s2buf, d2buf, hsbuf, hdbuf, s2t, d2t, sem0, sem1):
    c = lax.axis_index("c")
    s = lax.axis_index("s")
    cn = c * _N
    base = s * _DEC_EPT

    def chunk(e0, k, s2b, d2b, hsb, hdb):
        pltpu.sync_copy(src_hbm.at[pl.ds(e0, k)], s2b)
        pltpu.sync_copy(dst_hbm.at[pl.ds(e0, k)], d2b)
        _iota_add(s2b, s2b, cn)
        _iota_add(d2b, d2b, cn)
        cp0 = pltpu.async_copy(h2_hbm.at[s2b], hsb, sem0)
        cp1 = pltpu.async_copy(h2_hbm.at[d2b], hdb, sem1)
        cp0.wait()
        cp1.wait()

        def row(r, carry):
            for f in range(8):
                sl = pl.ds(f * _L, _L)
                hsb[r, sl] = jnp.abs(hsb[r, sl] - hdb[r, sl])
            return carry

        lax.fori_loop(0, k, row, 0)
        pltpu.sync_copy(hsb, out_hbm.at[c, pl.ds(e0, k)])

    def body(j, carry):
        chunk(base + j * _K, _K, s2buf, d2buf, hsbuf, hdbuf)
        return carry

    lax.fori_loop(0, _DEC_NCH, body, 0)
    tl = pl.ds(0, _DEC_TAIL)
    chunk(base + _DEC_NCH * _K, _DEC_TAIL, s2t, d2t,
          hsbuf.at[tl], hdbuf.at[tl])


@functools.cache
def _sc_kernels():
    f32, i32 = jnp.float32, jnp.int32
    mesh = plsc.VectorSubcoreMesh(
        core_axis_name="c", subcore_axis_name="s",
        num_cores=_SC_NC, num_subcores=_SC_NS)
    enc = pl.kernel(
        _sc_enc_body, mesh=mesh,
        out_type=jax.ShapeDtypeStruct((2, _N, 128), f32),
        scratch_types=[
            pltpu.VMEM((_K, 128), f32),
            pltpu.VMEM((_ENC_TAIL, 128), f32),
            pltpu.VMEM((_K,), i32),
            pltpu.VMEM((_K,), i32),
            pltpu.VMEM((_ENC_TAIL,), i32),
            pltpu.VMEM((_ENC_TAIL,), i32),
            pltpu.VMEM_SHARED((_N, 128), f32),
        ])
    msg = pl.kernel(
        _sc_msg_body, mesh=mesh,
        out_type=jax.ShapeDtypeStruct((2, _N, 128), f32),
        scratch_types=[
            pltpu.VMEM((_KM,), i32),
            pltpu.VMEM((_KM,), i32),
            pltpu.VMEM((_KM,), i32),
            pltpu.VMEM((_KM,), i32),
            pltpu.VMEM((_KM, 256), f32),
            pltpu.VMEM((_KM, 128), f32),
            pltpu.VMEM((_KM, 128), f32),
            pltpu.VMEM((_MSG_TAIL,), i32),
            pltpu.VMEM((_MSG_TAIL,), i32),
            pltpu.VMEM((_MSG_TAIL,), i32),
            pltpu.VMEM((_MSG_TAIL,), i32),
            pltpu.VMEM_SHARED((_N, 128), f32),
            pltpu.SemaphoreType.DMA,
            pltpu.SemaphoreType.DMA,
        ])
    dec = pl.kernel(
        _sc_dec_body, mesh=mesh,
        out_type=jax.ShapeDtypeStruct((2, _E, 128), f32),
        scratch_types=[
            pltpu.VMEM((_K,), i32),
            pltpu.VMEM((_K,), i32),
            pltpu.VMEM((_K, 128), f32),
            pltpu.VMEM((_K, 128), f32),
            pltpu.VMEM((_DEC_TAIL,), i32),
            pltpu.VMEM((_DEC_TAIL,), i32),
            pltpu.SemaphoreType.DMA,
            pltpu.SemaphoreType.DMA,
        ])
    return enc, msg, dec


def kernel(x, edge_index, edge_attr, batch, params):
    p = params
    f32 = jnp.float32
    src = edge_index[0]
    dst = edge_index[1]

    we_cat = jnp.concatenate(
        [jnp.concatenate([p['WE'][l][:, :128], p['WE'][l][:, 128:]], axis=1)
         for l in range(_NLAYERS)], axis=1)
    bex_cat = jnp.concatenate([p['bE'][l] for l in range(_NLAYERS)])[None]
    t, ex0, ex1, ex2 = _edge_enc(
        edge_attr, p['e2n_W'], p['e2n_b'][None], p['e2n_g'][None],
        p['e2n_be'][None], we_cat, bex_cat)
    exs = (ex0, ex1, ex2)

    z128 = jnp.zeros((_N, 128), f32)

    if _USE_SC:
        _enc_k, _msg_k, _dec_k = _sc_kernels()
        nfa = _enc_k(t, src, dst, z128)
    else:
        nf = jnp.zeros((_N, 128), f32).at[dst].add(t).at[src].add(t)
        deg = jnp.zeros((_N,), f32).at[src].add(1.0).at[dst].add(1.0)
        d128 = jnp.concatenate(
            [deg[:, None], jnp.zeros((_N, 127), f32)], axis=1)
        nfa = jnp.stack([nf, d128])

    h = _emb(x, nfa, p['emb_W'], p['emb_b'][None])

    for l in range(_NLAYERS):
        wa, wb, wc, wd = p['WA'][l], p['WB'][l], p['WC'][l], p['WD'][l]
        w_cat = jnp.concatenate(
            [wa[:, :128], wb[:, :128], wa[:, 128:], wb[:, 128:],
             wc[:, :128], wc[:, 128:], wd], axis=1)
        b_cat = jnp.concatenate(
            [p['bA'][l][:128], p['bB'][l][:128], p['bA'][l][128:],
             p['bB'][l][128:], p['bC'][l][:128], p['bC'][l][128:],
             p['bD'][l]])[None]
        ab3, ct3, dx = _node_mm(h, w_cat, b_cat)
        ab2 = ab3.reshape(2 * _N, 256)
        ct2 = ct3.reshape(2 * _N, 128)

        if _USE_SC:
            agg2 = _msg_k(ab2, ct2, exs[l], src, dst, z128)
        else:
            ax = jnp.concatenate([ab3[0, :, :128], ab3[1, :, :128]], axis=1)
            bx = jnp.concatenate([ab3[0, :, 128:], ab3[1, :, 128:]], axis=1)
            cx = jnp.concatenate([ct3[0], ct3[1]], axis=1)
            ex = jnp.concatenate([exs[l][0], exs[l][1]], axis=1)
            sig = jax.nn.sigmoid(bx[src] + cx[dst] + ex)
            agg = jnp.zeros((_N, _DH), f32).at[dst].add(ax[src] * sig)
            agg2 = jnp.stack([agg[:, :128], agg[:, 128:]])

        u, st = _bn_stats(agg2, dx, h)
        h, h2 = _bn_apply(u, st, p['bn_g'][l][None], p['bn_b'][l][None])

    if _USE_SC:
        dcat = _dec_k(h2.reshape(2 * _N, 128), src, dst)
    else:
        d = jnp.abs(h[src] - h[dst])
        dcat = jnp.stack([d[:, :128], d[:, 128:]])

    ep = _dec_mlp(dcat, p['dec_W1'], p['dec_b1'][None],
                  p['dec_W2'], p['dec_b2'][None, :])
    adj_pred = ep.reshape(_E)

    gs, gc = _pool(h, batch[:, None])
    class_logits = _cls(gs, gc, p['cls_W1'], p['cls_b1'][None],
                        p['cls_W2'], p['cls_b2'][None])

    return (adj_pred, class_logits, h)
